# Initial kernel scaffold; baseline (speedup 1.0000x reference)
import jax
import jax.numpy as jnp
from jax.experimental import pallas as pl


def kernel(node_representations, edges, edge_features, W1, b1, W2, W3, b3, W4):
    raise NotImplementedError("write your pallas kernel here")


# speedup vs baseline: 4.1226x; 4.1226x over previous
import functools

import jax
import jax.numpy as jnp
from jax import lax
from jax.experimental import pallas as pl
from jax.experimental.pallas import tpu as pltpu
from jax.experimental.pallas import tpu_sc as plsc

N_NODES = 10000
N_EDGES = 320000
D_FEAT = 128
D_EDGE = 16
L = 16
NC = 2
NS = 16
NW = NC * NS
CHUNK = 128
CH_PER_W = 79
E_PAD = NW * CH_PER_W * CHUNK
ACC_ROWS = 10240
DUMMY_ROW = 10000
ZROWS = ACC_ROWS // NS


def _proj_nodes_body(nodes_ref, w_ref, out_ref):
    out_ref[...] = jnp.dot(nodes_ref[...], w_ref[...],
                           preferred_element_type=jnp.float32)


def _proj_edges_body(ef_ref, w_ref, b_ref, out_ref):
    out_ref[...] = jnp.dot(ef_ref[...], w_ref[...],
                           preferred_element_type=jnp.float32) + b_ref[...]


def _update_body(nodes_ref, h0_ref, h1_ref, w2_ref, sel_ref, w3a_ref,
                 w3b_ref, b3_ref, w4_ref, out_ref):
    hs = h0_ref[...] + h1_ref[...]
    cnt = jnp.maximum(jnp.dot(hs, sel_ref[...],
                              preferred_element_type=jnp.float32), 1.0)
    agg = jnp.dot(hs, w2_ref[...], preferred_element_type=jnp.float32) / cnt
    u = jnp.dot(nodes_ref[...], w3a_ref[...],
                preferred_element_type=jnp.float32)
    u = u + jnp.dot(agg, w3b_ref[...], preferred_element_type=jnp.float32)
    u = jnp.maximum(u + b3_ref[...], 0.0)
    out_ref[...] = jnp.dot(u, w4_ref[...], preferred_element_type=jnp.float32)


def _sc_edge_body(src_hbm, dst_hbm, ep_hbm, p_hbm, out_hbm,
                  src_v, dst_v, ep_v, rows_v, zbuf, acc, sem):
    c = lax.axis_index("c")
    s = lax.axis_index("s")
    wid = s * NC + c

    def zero_row(i, carry):
        zbuf[i] = jnp.zeros((L,), jnp.float32)
        return carry
    lax.fori_loop(0, ZROWS, zero_row, 0)
    pltpu.sync_copy(zbuf, acc.at[pl.ds(s * ZROWS, ZROWS)])
    plsc.subcore_barrier()

    pltpu.sync_copy(src_hbm.at[wid], src_v)
    pltpu.sync_copy(dst_hbm.at[wid], dst_v)

    base = wid * CH_PER_W * CHUNK

    def chunk(j, carry):
        pltpu.sync_copy(ep_hbm.at[pl.ds(base + j * CHUNK, CHUNK)], ep_v)
        pltpu.async_copy(p_hbm.at[src_v.at[j]], rows_v, sem).wait()

        def edge(i, c2):
            rows_v[i] = jnp.maximum(rows_v[i] + ep_v[i], 0.0)
            return c2
        lax.fori_loop(0, CHUNK, edge, 0, unroll=8)
        pltpu.sync_copy(rows_v, acc.at[dst_v.at[j]], add=True)
        return carry

    lax.fori_loop(0, CH_PER_W, chunk, 0)
    plsc.subcore_barrier()

    pltpu.sync_copy(acc.at[pl.ds(s * ZROWS, ZROWS)],
                    out_hbm.at[c, pl.ds(s * ZROWS, ZROWS)])


def _make_sc_kernel():
    mesh = plsc.VectorSubcoreMesh(core_axis_name="c", subcore_axis_name="s",
                                  num_cores=NC, num_subcores=NS)
    return pl.kernel(
        _sc_edge_body,
        out_type=jax.ShapeDtypeStruct((NC, ACC_ROWS, L), jnp.float32),
        mesh=mesh,
        compiler_params=pltpu.CompilerParams(use_tc_tiling_on_sc=False),
        scratch_types=[
            pltpu.VMEM((CH_PER_W, CHUNK), jnp.int32),
            pltpu.VMEM((CH_PER_W, CHUNK), jnp.int32),
            pltpu.VMEM((CHUNK, L), jnp.float32),
            pltpu.VMEM((CHUNK, L), jnp.float32),
            pltpu.VMEM((ZROWS, L), jnp.float32),
            pltpu.VMEM_SHARED((ACC_ROWS, L), jnp.float32),
            pltpu.SemaphoreType.DMA,
        ],
    )


def kernel(node_representations, edges, edge_features, W1, b1, W2, W3, b3, W4):
    nodes = node_representations
    src = edges[0].astype(jnp.int32)
    dst = edges[1].astype(jnp.int32)
    pad = E_PAD - N_EDGES
    src_p = jnp.pad(src, (0, pad)).reshape(NW, CH_PER_W, CHUNK)
    dst_p = jnp.pad(dst, (0, pad), constant_values=DUMMY_ROW
                    ).reshape(NW, CH_PER_W, CHUNK)
    ef_p = jnp.pad(edge_features, ((0, pad), (0, 0)))

    W1a = jnp.pad(W1[:D_EDGE], ((0, 0), (0, L - 10)))
    W1b = jnp.pad(W1[D_EDGE:], ((0, 0), (0, L - 10)))
    b1p = jnp.concatenate([b1, jnp.ones((1,), jnp.float32),
                           jnp.zeros((L - 11,), jnp.float32)]).reshape(1, L)
    W2p = jnp.pad(W2, ((0, L - 10), (0, L - 10)))
    sel = jnp.zeros((L, 1), jnp.float32).at[10, 0].set(1.0)
    W3a = W3[:D_FEAT]
    W3b = jnp.pad(W3[D_FEAT:], ((0, L - 10), (0, 0)))
    b3r = b3.reshape(1, -1)

    proj_nodes = pl.pallas_call(
        _proj_nodes_body,
        out_shape=jax.ShapeDtypeStruct((N_NODES, L), jnp.float32),
    )
    P = proj_nodes(nodes, W1b)

    eblk = E_PAD // 16
    proj_edges = pl.pallas_call(
        _proj_edges_body,
        grid=(16,),
        in_specs=[
            pl.BlockSpec((eblk, D_EDGE), lambda i: (i, 0)),
            pl.BlockSpec((D_EDGE, L), lambda i: (0, 0)),
            pl.BlockSpec((1, L), lambda i: (0, 0)),
        ],
        out_specs=pl.BlockSpec((eblk, L), lambda i: (i, 0)),
        out_shape=jax.ShapeDtypeStruct((E_PAD, L), jnp.float32),
    )
    Ep = proj_edges(ef_p, W1a, b1p)

    parts = _make_sc_kernel()(src_p, dst_p, Ep, P)

    update = pl.pallas_call(
        _update_body,
        out_shape=jax.ShapeDtypeStruct((N_NODES, 10), jnp.float32),
    )
    return update(nodes, parts[0, :N_NODES], parts[1, :N_NODES], W2p, sel,
                  W3a, W3b, b3r, W4)


# speedup vs baseline: 5.4898x; 1.3316x over previous
import jax
import jax.numpy as jnp
from jax import lax
from jax.experimental import pallas as pl
from jax.experimental.pallas import tpu as pltpu
from jax.experimental.pallas import tpu_sc as plsc

N_NODES = 10000
N_EDGES = 320000
D_FEAT = 128
D_EDGE = 16
L = 16
NC = 2
NS = 16
NW = NC * NS
CHUNK = 128
NCH = N_EDGES // CHUNK
NCH_MAIN = NCH // NW
NCH_TAIL = NCH - NCH_MAIN * NW
CHB = NW * (NCH_MAIN + 1)
ACC_ROWS = 10240
ZROWS = ACC_ROWS // NS


def _proj_nodes_body(nodes_ref, w_ref, out_ref):
    out_ref[...] = jnp.dot(nodes_ref[...], w_ref[...],
                           preferred_element_type=jnp.float32)


def _proj_edges_body(ef_ref, w_ref, b_ref, out_ref):
    out_ref[...] = jnp.dot(ef_ref[...], w_ref[...],
                           preferred_element_type=jnp.float32) + b_ref[...]


def _update_body(nodes_ref, h0_ref, h1_ref, w2_ref, sel_ref, w3a_ref,
                 w3b_ref, b3_ref, w4_ref, out_ref):
    hs = h0_ref[...] + h1_ref[...]
    cnt = jnp.maximum(jnp.dot(hs, sel_ref[...],
                              preferred_element_type=jnp.float32), 1.0)
    agg = jnp.dot(hs, w2_ref[...], preferred_element_type=jnp.float32) / cnt
    u = jnp.dot(nodes_ref[...], w3a_ref[...],
                preferred_element_type=jnp.float32)
    u = u + jnp.dot(agg, w3b_ref[...], preferred_element_type=jnp.float32)
    u = jnp.maximum(u + b3_ref[...], 0.0)
    out_ref[...] = jnp.dot(u, w4_ref[...], preferred_element_type=jnp.float32)


def _sc_edge_body(src_hbm, dst_hbm, ep_hbm, p_hbm, out_hbm,
                  src_v, dst_v, ep0, ep1, rows0, rows1, zbuf, acc,
                  gsem0, gsem1, esem0, esem1):
    c = lax.axis_index("c")
    s = lax.axis_index("s")
    wid = s * NC + c

    def zero_row(i, carry):
        zbuf[i] = jnp.zeros((L,), jnp.float32)
        return carry
    lax.fori_loop(0, ZROWS, zero_row, 0)
    pltpu.sync_copy(zbuf, acc.at[pl.ds(s * ZROWS, ZROWS)])
    plsc.subcore_barrier()

    pltpu.sync_copy(src_hbm.at[pl.ds(wid * NCH_MAIN, NCH_MAIN)],
                    src_v.at[pl.ds(0, NCH_MAIN)])
    pltpu.sync_copy(dst_hbm.at[pl.ds(wid * NCH_MAIN, NCH_MAIN)],
                    dst_v.at[pl.ds(0, NCH_MAIN)])

    @pl.when(wid < NCH_TAIL)
    def _():
        pltpu.sync_copy(src_hbm.at[NW * NCH_MAIN + wid], src_v.at[NCH_MAIN])
        pltpu.sync_copy(dst_hbm.at[NW * NCH_MAIN + wid], dst_v.at[NCH_MAIN])

    base = wid * NCH_MAIN * CHUNK

    def fire(j, rows_ref, ep_ref, gsem, esem):
        pltpu.async_copy(ep_hbm.at[pl.ds(base + j * CHUNK, CHUNK)],
                         ep_ref, esem)
        pltpu.async_copy(p_hbm.at[src_v.at[j]], rows_ref, gsem)

    def wait_slot(rows_ref, ep_ref, gsem, esem):
        pltpu.make_async_copy(ep_hbm.at[pl.ds(0, CHUNK)], ep_ref, esem).wait()
        pltpu.make_async_copy(ep_hbm.at[pl.ds(0, CHUNK)], rows_ref,
                              gsem).wait()

    def compute_scatter(j, rows_ref, ep_ref):
        def edge(i, c2):
            rows_ref[i] = jnp.maximum(rows_ref[i] + ep_ref[i], 0.0)
            return c2
        lax.fori_loop(0, CHUNK, edge, 0, unroll=8)
        pltpu.sync_copy(rows_ref, acc.at[dst_v.at[j]], add=True)

    fire(0, rows0, ep0, gsem0, esem0)

    def body(j, carry):
        def proc(rq, eq, gs, es, ro, eo, go, eso):
            wait_slot(rq, eq, gs, es)

            @pl.when(j + 1 < NCH_MAIN)
            def _():
                fire(j + 1, ro, eo, go, eso)
            compute_scatter(j, rq, eq)

        @pl.when(j % 2 == 0)
        def _():
            proc(rows0, ep0, gsem0, esem0, rows1, ep1, gsem1, esem1)

        @pl.when(j % 2 == 1)
        def _():
            proc(rows1, ep1, gsem1, esem1, rows0, ep0, gsem0, esem0)
        return carry

    lax.fori_loop(0, NCH_MAIN, body, 0)

    @pl.when(wid < NCH_TAIL)
    def _():
        tail_base = (NW * NCH_MAIN + wid) * CHUNK
        pltpu.sync_copy(ep_hbm.at[pl.ds(tail_base, CHUNK)], ep0)
        pltpu.async_copy(p_hbm.at[src_v.at[NCH_MAIN]], rows0, gsem0).wait()
        compute_scatter(NCH_MAIN, rows0, ep0)

    plsc.subcore_barrier()

    pltpu.sync_copy(acc.at[pl.ds(s * ZROWS, ZROWS)],
                    out_hbm.at[c, pl.ds(s * ZROWS, ZROWS)])


def _make_sc_kernel():
    mesh = plsc.VectorSubcoreMesh(core_axis_name="c", subcore_axis_name="s",
                                  num_cores=NC, num_subcores=NS)
    return pl.kernel(
        _sc_edge_body,
        out_type=jax.ShapeDtypeStruct((NC, ACC_ROWS, L), jnp.float32),
        mesh=mesh,
        compiler_params=pltpu.CompilerParams(use_tc_tiling_on_sc=False),
        scratch_types=[
            pltpu.VMEM((NCH_MAIN + 1, CHUNK), jnp.int32),
            pltpu.VMEM((NCH_MAIN + 1, CHUNK), jnp.int32),
            pltpu.VMEM((CHUNK, L), jnp.float32),
            pltpu.VMEM((CHUNK, L), jnp.float32),
            pltpu.VMEM((CHUNK, L), jnp.float32),
            pltpu.VMEM((CHUNK, L), jnp.float32),
            pltpu.VMEM((ZROWS, L), jnp.float32),
            pltpu.VMEM_SHARED((ACC_ROWS, L), jnp.float32),
            pltpu.SemaphoreType.DMA,
            pltpu.SemaphoreType.DMA,
            pltpu.SemaphoreType.DMA,
            pltpu.SemaphoreType.DMA,
        ],
    )


def kernel(node_representations, edges, edge_features, W1, b1, W2, W3, b3, W4):
    nodes = node_representations
    src = edges[0].astype(jnp.int32)
    dst = edges[1].astype(jnp.int32)
    pad = CHB * CHUNK - N_EDGES
    src_p = jnp.pad(src, (0, pad)).reshape(CHB, CHUNK)
    dst_p = jnp.pad(dst, (0, pad)).reshape(CHB, CHUNK)

    W1a = jnp.pad(W1[:D_EDGE], ((0, 0), (0, L - 10)))
    W1b = jnp.pad(W1[D_EDGE:], ((0, 0), (0, L - 10)))
    b1p = jnp.concatenate([b1, jnp.ones((1,), jnp.float32),
                           jnp.zeros((L - 11,), jnp.float32)]).reshape(1, L)
    W2p = jnp.pad(W2, ((0, L - 10), (0, L - 10)))
    sel = jnp.zeros((L, 1), jnp.float32).at[10, 0].set(1.0)
    W3a = W3[:D_FEAT]
    W3b = jnp.pad(W3[D_FEAT:], ((0, L - 10), (0, 0)))
    b3r = b3.reshape(1, -1)

    proj_nodes = pl.pallas_call(
        _proj_nodes_body,
        out_shape=jax.ShapeDtypeStruct((N_NODES, L), jnp.float32),
    )
    P = proj_nodes(nodes, W1b)

    eblk = N_EDGES // 16
    proj_edges = pl.pallas_call(
        _proj_edges_body,
        grid=(16,),
        in_specs=[
            pl.BlockSpec((eblk, D_EDGE), lambda i: (i, 0)),
            pl.BlockSpec((D_EDGE, L), lambda i: (0, 0)),
            pl.BlockSpec((1, L), lambda i: (0, 0)),
        ],
        out_specs=pl.BlockSpec((eblk, L), lambda i: (i, 0)),
        out_shape=jax.ShapeDtypeStruct((N_EDGES, L), jnp.float32),
    )
    Ep = proj_edges(edge_features, W1a, b1p)

    parts = _make_sc_kernel()(src_p, dst_p, Ep, P)

    update = pl.pallas_call(
        _update_body,
        out_shape=jax.ShapeDtypeStruct((N_NODES, 10), jnp.float32),
    )
    return update(nodes, parts[0, :N_NODES], parts[1, :N_NODES], W2p, sel,
                  W3a, W3b, b3r, W4)


# speedup vs baseline: 7.8022x; 1.4212x over previous
import jax
import jax.numpy as jnp
from jax import lax
from jax.experimental import pallas as pl
from jax.experimental.pallas import tpu as pltpu
from jax.experimental.pallas import tpu_sc as plsc

N_NODES = 10000
N_EDGES = 320000
D_FEAT = 128
D_EDGE = 16
L = 16
NC = 2
NS = 16
NW = NC * NS
CHUNK = 128
NCH = N_EDGES // CHUNK
NCH_MAIN = NCH // NW
NCH_TAIL = NCH - NCH_MAIN * NW
CHB = NW * (NCH_MAIN + 1)
ACC_ROWS = 10240
ZROWS = ACC_ROWS // NS


def _proj_nodes_body(nodes_ref, w_ref, out_ref):
    out_ref[...] = jnp.dot(nodes_ref[...], w_ref[...],
                           preferred_element_type=jnp.float32)


CPB = 25


def _proj_edges_body(eft_ref, w_ref, b_ref, out_ref):
    ept = jax.lax.dot_general(w_ref[...], eft_ref[...],
                              (((0,), (0,)), ((), ())),
                              preferred_element_type=jnp.float32) + b_ref[...]
    for i in range(CPB):
        out_ref[i] = ept[:, CHUNK * i:CHUNK * (i + 1)]


def _update_body(nodes_ref, h0_ref, h1_ref, w2_ref, sel_ref, w3a_ref,
                 w3b_ref, b3_ref, w4_ref, out_ref):
    hs = h0_ref[...] + h1_ref[...]
    cnt = jnp.maximum(jnp.dot(hs, sel_ref[...],
                              preferred_element_type=jnp.float32), 1.0)
    agg = jnp.dot(hs, w2_ref[...], preferred_element_type=jnp.float32) / cnt
    u = jnp.dot(nodes_ref[...], w3a_ref[...],
                preferred_element_type=jnp.float32)
    u = u + jnp.dot(agg, w3b_ref[...], preferred_element_type=jnp.float32)
    u = jnp.maximum(u + b3_ref[...], 0.0)
    out_ref[...] = jnp.dot(u, w4_ref[...], preferred_element_type=jnp.float32)


def _sc_edge_body(src_hbm, dst_hbm, ep_hbm, p_hbm, out_hbm,
                  src_v, dst_v, ep0, ep1, rows0, rows1, zbuf, acc,
                  gsem0, gsem1, esem0, esem1):
    c = lax.axis_index("c")
    s = lax.axis_index("s")
    wid = s * NC + c

    def zero_row(i, carry):
        zbuf[i] = jnp.zeros((L,), jnp.float32)
        return carry
    lax.fori_loop(0, ZROWS, zero_row, 0)
    pltpu.sync_copy(zbuf, acc.at[pl.ds(s * ZROWS, ZROWS)])
    plsc.subcore_barrier()

    pltpu.sync_copy(src_hbm.at[pl.ds(wid * NCH_MAIN, NCH_MAIN)],
                    src_v.at[pl.ds(0, NCH_MAIN)])
    pltpu.sync_copy(dst_hbm.at[pl.ds(wid * NCH_MAIN, NCH_MAIN)],
                    dst_v.at[pl.ds(0, NCH_MAIN)])

    @pl.when(wid < NCH_TAIL)
    def _():
        pltpu.sync_copy(src_hbm.at[NW * NCH_MAIN + wid], src_v.at[NCH_MAIN])
        pltpu.sync_copy(dst_hbm.at[NW * NCH_MAIN + wid], dst_v.at[NCH_MAIN])

    base_ch = wid * NCH_MAIN
    rows16 = lax.iota(jnp.int32, L)

    def fire(j, rows_ref, ep_ref, gsem, esem):
        pltpu.async_copy(ep_hbm.at[base_ch + j], ep_ref, esem)
        pltpu.async_copy(p_hbm.at[src_v.at[j]], rows_ref, gsem)

    def wait_slot(rows_ref, ep_ref, gsem, esem):
        pltpu.make_async_copy(ep_hbm.at[0], ep_ref, esem).wait()
        pltpu.make_async_copy(p_hbm.at[pl.ds(0, CHUNK)], rows_ref,
                              gsem).wait()

    def compute_scatter(j, rows_ref, ep_ref):
        def edge(i, c2):
            col = jnp.full((L,), i, jnp.int32)
            epcol = plsc.load_gather(ep_ref, [rows16, col])
            rows_ref[i] = jnp.maximum(rows_ref[i] + epcol, 0.0)
            return c2
        lax.fori_loop(0, CHUNK, edge, 0, unroll=8)
        pltpu.sync_copy(rows_ref, acc.at[dst_v.at[j]], add=True)

    fire(0, rows0, ep0, gsem0, esem0)

    def body(j, carry):
        def proc(rq, eq, gs, es, ro, eo, go, eso):
            wait_slot(rq, eq, gs, es)

            @pl.when(j + 1 < NCH_MAIN)
            def _():
                fire(j + 1, ro, eo, go, eso)
            compute_scatter(j, rq, eq)

        @pl.when(j % 2 == 0)
        def _():
            proc(rows0, ep0, gsem0, esem0, rows1, ep1, gsem1, esem1)

        @pl.when(j % 2 == 1)
        def _():
            proc(rows1, ep1, gsem1, esem1, rows0, ep0, gsem0, esem0)
        return carry

    lax.fori_loop(0, NCH_MAIN, body, 0)

    @pl.when(wid < NCH_TAIL)
    def _():
        pltpu.sync_copy(ep_hbm.at[NW * NCH_MAIN + wid], ep0)
        pltpu.async_copy(p_hbm.at[src_v.at[NCH_MAIN]], rows0, gsem0).wait()
        compute_scatter(NCH_MAIN, rows0, ep0)

    plsc.subcore_barrier()

    pltpu.sync_copy(acc.at[pl.ds(s * ZROWS, ZROWS)],
                    out_hbm.at[c, pl.ds(s * ZROWS, ZROWS)])


def _make_sc_kernel():
    mesh = plsc.VectorSubcoreMesh(core_axis_name="c", subcore_axis_name="s",
                                  num_cores=NC, num_subcores=NS)
    return pl.kernel(
        _sc_edge_body,
        out_type=jax.ShapeDtypeStruct((NC, ACC_ROWS, L), jnp.float32),
        mesh=mesh,
        compiler_params=pltpu.CompilerParams(use_tc_tiling_on_sc=False,
                                             needs_layout_passes=False),
        scratch_types=[
            pltpu.VMEM((NCH_MAIN + 1, CHUNK), jnp.int32),
            pltpu.VMEM((NCH_MAIN + 1, CHUNK), jnp.int32),
            pltpu.VMEM((L, CHUNK), jnp.float32),
            pltpu.VMEM((L, CHUNK), jnp.float32),
            pltpu.VMEM((CHUNK, L), jnp.float32),
            pltpu.VMEM((CHUNK, L), jnp.float32),
            pltpu.VMEM((ZROWS, L), jnp.float32),
            pltpu.VMEM_SHARED((ACC_ROWS, L), jnp.float32),
            pltpu.SemaphoreType.DMA,
            pltpu.SemaphoreType.DMA,
            pltpu.SemaphoreType.DMA,
            pltpu.SemaphoreType.DMA,
        ],
    )


def kernel(node_representations, edges, edge_features, W1, b1, W2, W3, b3, W4):
    nodes = node_representations
    src = edges[0].astype(jnp.int32)
    dst = edges[1].astype(jnp.int32)
    pad = CHB * CHUNK - N_EDGES
    src_p = jnp.pad(src, (0, pad)).reshape(CHB, CHUNK)
    dst_p = jnp.pad(dst, (0, pad)).reshape(CHB, CHUNK)

    W1a = jnp.pad(W1[:D_EDGE], ((0, 0), (0, L - 10)))
    W1b = jnp.pad(W1[D_EDGE:], ((0, 0), (0, L - 10)))
    b1p = jnp.concatenate([b1, jnp.ones((1,), jnp.float32),
                           jnp.zeros((L - 11,), jnp.float32)]).reshape(L, 1)
    W2p = jnp.pad(W2, ((0, L - 10), (0, L - 10)))
    sel = jnp.zeros((L, 1), jnp.float32).at[10, 0].set(1.0)
    W3a = W3[:D_FEAT]
    W3b = jnp.pad(W3[D_FEAT:], ((0, L - 10), (0, 0)))
    b3r = b3.reshape(1, -1)

    proj_nodes = pl.pallas_call(
        _proj_nodes_body,
        out_shape=jax.ShapeDtypeStruct((N_NODES, L), jnp.float32),
    )
    P = proj_nodes(nodes, W1b)

    eft = edge_features.T
    proj_edges = pl.pallas_call(
        _proj_edges_body,
        grid=(NCH // CPB,),
        in_specs=[
            pl.BlockSpec((D_EDGE, CPB * CHUNK), lambda i: (0, i)),
            pl.BlockSpec((D_EDGE, L), lambda i: (0, 0)),
            pl.BlockSpec((L, 1), lambda i: (0, 0)),
        ],
        out_specs=pl.BlockSpec((CPB, L, CHUNK), lambda i: (i, 0, 0)),
        out_shape=jax.ShapeDtypeStruct((NCH, L, CHUNK), jnp.float32),
    )
    Ep = proj_edges(eft, W1a, b1p)

    parts = _make_sc_kernel()(src_p, dst_p, Ep, P)

    update = pl.pallas_call(
        _update_body,
        out_shape=jax.ShapeDtypeStruct((N_NODES, 10), jnp.float32),
    )
    return update(nodes, parts[0, :N_NODES], parts[1, :N_NODES], W2p, sel,
                  W3a, W3b, b3r, W4)


# speedup vs baseline: 9.0828x; 1.1641x over previous
import jax
import jax.numpy as jnp
from jax import lax
from jax.experimental import pallas as pl
from jax.experimental.pallas import tpu as pltpu
from jax.experimental.pallas import tpu_sc as plsc

N_NODES = 10000
N_EDGES = 320000
D_FEAT = 128
D_EDGE = 16
L = 16
NC = 2
NS = 16
NW = NC * NS
CHUNK = 128
NCH = N_EDGES // CHUNK
NCH_MAIN = NCH // NW
NCH_TAIL = NCH - NCH_MAIN * NW
CHB = NW * (NCH_MAIN + 1)
ACC_ROWS = 10240
ZROWS = ACC_ROWS // NS


def _proj_nodes_body(nodes_ref, w_ref, out_ref):
    out_ref[...] = jnp.dot(nodes_ref[...], w_ref[...],
                           preferred_element_type=jnp.float32)


CPB = 125


def _proj_edges_body(eft_ref, w_ref, b_ref, out_ref):
    ept = jax.lax.dot_general(w_ref[...], eft_ref[...],
                              (((0,), (0,)), ((), ())),
                              preferred_element_type=jnp.float32) + b_ref[...]
    for i in range(CPB):
        out_ref[i] = ept[:, CHUNK * i:CHUNK * (i + 1)]


def _update_body(nodes_ref, h0_ref, h1_ref, w2_ref, sel_ref, w3a_ref,
                 w3b_ref, b3_ref, w4_ref, out_ref):
    hs = h0_ref[...] + h1_ref[...]
    cnt = jnp.maximum(jnp.dot(hs, sel_ref[...],
                              preferred_element_type=jnp.float32), 1.0)
    agg = jnp.dot(hs, w2_ref[...], preferred_element_type=jnp.float32) / cnt
    u = jnp.dot(nodes_ref[...], w3a_ref[...],
                preferred_element_type=jnp.float32)
    u = u + jnp.dot(agg, w3b_ref[...], preferred_element_type=jnp.float32)
    u = jnp.maximum(u + b3_ref[...], 0.0)
    out_ref[...] = jnp.dot(u, w4_ref[...], preferred_element_type=jnp.float32)


def _sc_edge_body(src_hbm, dst_hbm, ep_hbm, p_hbm, out_hbm,
                  src_v, dst_v, ep0, ep1, rows0, rows1, zbuf, acc,
                  gsem0, gsem1, esem0, esem1):
    c = lax.axis_index("c")
    s = lax.axis_index("s")
    wid = s * NC + c

    def zero_row(i, carry):
        zbuf[i] = jnp.zeros((L,), jnp.float32)
        return carry
    lax.fori_loop(0, ZROWS, zero_row, 0)
    pltpu.sync_copy(zbuf, acc.at[pl.ds(s * ZROWS, ZROWS)])
    plsc.subcore_barrier()

    pltpu.sync_copy(src_hbm.at[pl.ds(wid * NCH_MAIN, NCH_MAIN)],
                    src_v.at[pl.ds(0, NCH_MAIN)])
    pltpu.sync_copy(dst_hbm.at[pl.ds(wid * NCH_MAIN, NCH_MAIN)],
                    dst_v.at[pl.ds(0, NCH_MAIN)])

    @pl.when(wid < NCH_TAIL)
    def _():
        pltpu.sync_copy(src_hbm.at[NW * NCH_MAIN + wid], src_v.at[NCH_MAIN])
        pltpu.sync_copy(dst_hbm.at[NW * NCH_MAIN + wid], dst_v.at[NCH_MAIN])

    base_ch = wid * NCH_MAIN
    lanes128 = lax.iota(jnp.int32, L) * CHUNK

    def fire(j, rows_ref, ep_ref, gsem, esem):
        pltpu.async_copy(ep_hbm.at[base_ch + j], ep_ref, esem)
        pltpu.async_copy(p_hbm.at[src_v.at[j]], rows_ref, gsem)

    def wait_slot(rows_ref, ep_ref, gsem, esem):
        pltpu.make_async_copy(ep_hbm.at[0], ep_ref, esem).wait()
        pltpu.make_async_copy(p_hbm.at[pl.ds(0, CHUNK)], rows_ref,
                              gsem).wait()

    def compute_scatter(j, rows_ref, ep_ref):
        def edge(i, c2):
            epcol = plsc.load_gather(ep_ref, [lanes128 + i])
            rows_ref[i] = jnp.maximum(rows_ref[i] + epcol, 0.0)
            return c2
        lax.fori_loop(0, CHUNK, edge, 0, unroll=8)
        pltpu.sync_copy(rows_ref, acc.at[dst_v.at[j]], add=True)

    fire(0, rows0, ep0, gsem0, esem0)

    def body(j, carry):
        def proc(rq, eq, gs, es, ro, eo, go, eso):
            wait_slot(rq, eq, gs, es)

            @pl.when(j + 1 < NCH_MAIN)
            def _():
                fire(j + 1, ro, eo, go, eso)
            compute_scatter(j, rq, eq)

        @pl.when(j % 2 == 0)
        def _():
            proc(rows0, ep0, gsem0, esem0, rows1, ep1, gsem1, esem1)

        @pl.when(j % 2 == 1)
        def _():
            proc(rows1, ep1, gsem1, esem1, rows0, ep0, gsem0, esem0)
        return carry

    lax.fori_loop(0, NCH_MAIN, body, 0)

    @pl.when(wid < NCH_TAIL)
    def _():
        pltpu.sync_copy(ep_hbm.at[NW * NCH_MAIN + wid], ep0)
        pltpu.async_copy(p_hbm.at[src_v.at[NCH_MAIN]], rows0, gsem0).wait()
        compute_scatter(NCH_MAIN, rows0, ep0)

    plsc.subcore_barrier()

    pltpu.sync_copy(acc.at[pl.ds(s * ZROWS, ZROWS)],
                    out_hbm.at[c, pl.ds(s * ZROWS, ZROWS)])


def _make_sc_kernel():
    mesh = plsc.VectorSubcoreMesh(core_axis_name="c", subcore_axis_name="s",
                                  num_cores=NC, num_subcores=NS)
    return pl.kernel(
        _sc_edge_body,
        out_type=jax.ShapeDtypeStruct((NC, ACC_ROWS, L), jnp.float32),
        mesh=mesh,
        compiler_params=pltpu.CompilerParams(use_tc_tiling_on_sc=False,
                                             needs_layout_passes=False),
        scratch_types=[
            pltpu.VMEM((NCH_MAIN + 1, CHUNK), jnp.int32),
            pltpu.VMEM((NCH_MAIN + 1, CHUNK), jnp.int32),
            pltpu.VMEM((L * CHUNK,), jnp.float32),
            pltpu.VMEM((L * CHUNK,), jnp.float32),
            pltpu.VMEM((CHUNK, L), jnp.float32),
            pltpu.VMEM((CHUNK, L), jnp.float32),
            pltpu.VMEM((ZROWS, L), jnp.float32),
            pltpu.VMEM_SHARED((ACC_ROWS, L), jnp.float32),
            pltpu.SemaphoreType.DMA,
            pltpu.SemaphoreType.DMA,
            pltpu.SemaphoreType.DMA,
            pltpu.SemaphoreType.DMA,
        ],
    )


def kernel(node_representations, edges, edge_features, W1, b1, W2, W3, b3, W4):
    nodes = node_representations
    src = edges[0].astype(jnp.int32)
    dst = edges[1].astype(jnp.int32)
    pad = CHB * CHUNK - N_EDGES
    src_p = jnp.pad(src, (0, pad)).reshape(CHB, CHUNK)
    dst_p = jnp.pad(dst, (0, pad)).reshape(CHB, CHUNK)

    W1a = jnp.pad(W1[:D_EDGE], ((0, 0), (0, L - 10)))
    W1b = jnp.pad(W1[D_EDGE:], ((0, 0), (0, L - 10)))
    b1p = jnp.concatenate([b1, jnp.ones((1,), jnp.float32),
                           jnp.zeros((L - 11,), jnp.float32)]).reshape(L, 1)
    W2p = jnp.pad(W2, ((0, L - 10), (0, L - 10)))
    sel = jnp.zeros((L, 1), jnp.float32).at[10, 0].set(1.0)
    W3a = W3[:D_FEAT]
    W3b = jnp.pad(W3[D_FEAT:], ((0, L - 10), (0, 0)))
    b3r = b3.reshape(1, -1)

    proj_nodes = pl.pallas_call(
        _proj_nodes_body,
        out_shape=jax.ShapeDtypeStruct((N_NODES, L), jnp.float32),
    )
    P = proj_nodes(nodes, W1b)

    eft = edge_features.T
    proj_edges = pl.pallas_call(
        _proj_edges_body,
        grid=(NCH // CPB,),
        in_specs=[
            pl.BlockSpec((D_EDGE, CPB * CHUNK), lambda i: (0, i)),
            pl.BlockSpec((D_EDGE, L), lambda i: (0, 0)),
            pl.BlockSpec((L, 1), lambda i: (0, 0)),
        ],
        out_specs=pl.BlockSpec((CPB, L, CHUNK), lambda i: (i, 0, 0)),
        out_shape=jax.ShapeDtypeStruct((NCH, L, CHUNK), jnp.float32),
    )
    Ep = proj_edges(eft, W1a, b1p)

    parts = _make_sc_kernel()(src_p, dst_p, Ep.reshape(NCH, L * CHUNK), P)

    update = pl.pallas_call(
        _update_body,
        out_shape=jax.ShapeDtypeStruct((N_NODES, 10), jnp.float32),
    )
    return update(nodes, parts[0, :N_NODES], parts[1, :N_NODES], W2p, sel,
                  W3a, W3b, b3r, W4)


# speedup vs baseline: 12.7896x; 1.4081x over previous
import jax
import jax.numpy as jnp
from jax import lax
from jax.experimental import pallas as pl
from jax.experimental.pallas import tpu as pltpu
from jax.experimental.pallas import tpu_sc as plsc

N_NODES = 10000
N_EDGES = 320000
D_FEAT = 128
D_EDGE = 16
L = 16
NC = 2
NS = 16
NW = NC * NS
CHUNK = 128
NCH = N_EDGES // CHUNK
NCH_MAIN = NCH // NW
NCH_TAIL = NCH - NCH_MAIN * NW
CHB = NW * (NCH_MAIN + 1)
ACC_ROWS = 10240
ZROWS = ACC_ROWS // NS


def _proj_nodes_body(nodes_ref, w_ref, out_ref):
    out_ref[...] = jnp.dot(nodes_ref[...], w_ref[...],
                           preferred_element_type=jnp.float32)


CPB = 125


def _proj_edges_body(eft_ref, w_ref, b_ref, out_ref):
    ept = jax.lax.dot_general(w_ref[...], eft_ref[...],
                              (((0,), (0,)), ((), ())),
                              preferred_element_type=jnp.float32) + b_ref[...]
    for i in range(CPB):
        out_ref[i] = ept[:, CHUNK * i:CHUNK * (i + 1)]


def _update_body(nodes_ref, h0_ref, h1_ref, w2_ref, sel_ref, w3a_ref,
                 w3b_ref, b3_ref, w4_ref, out_ref):
    hs = h0_ref[...] + h1_ref[...]
    cnt = jnp.maximum(jnp.dot(hs, sel_ref[...],
                              preferred_element_type=jnp.float32), 1.0)
    agg = jnp.dot(hs, w2_ref[...], preferred_element_type=jnp.float32) / cnt
    u = jnp.dot(nodes_ref[...], w3a_ref[...],
                preferred_element_type=jnp.float32)
    u = u + jnp.dot(agg, w3b_ref[...], preferred_element_type=jnp.float32)
    u = jnp.maximum(u + b3_ref[...], 0.0)
    out_ref[...] = jnp.dot(u, w4_ref[...], preferred_element_type=jnp.float32)


def _sc_edge_body(src_hbm, dst_hbm, ep_hbm, p_hbm, out_hbm,
                  src_v, dst_v, ep0, ep1, ept0, ept1, rows0, rows1, zbuf, acc,
                  gsem0, gsem1, esem0, esem1):
    c = lax.axis_index("c")
    s = lax.axis_index("s")
    wid = s * NC + c

    def zero_row(i, carry):
        zbuf[i] = jnp.zeros((L,), jnp.float32)
        return carry
    lax.fori_loop(0, ZROWS, zero_row, 0)
    pltpu.sync_copy(zbuf, acc.at[pl.ds(s * ZROWS, ZROWS)])
    plsc.subcore_barrier()

    pltpu.sync_copy(src_hbm.at[pl.ds(wid * NCH_MAIN, NCH_MAIN)],
                    src_v.at[pl.ds(0, NCH_MAIN)])
    pltpu.sync_copy(dst_hbm.at[pl.ds(wid * NCH_MAIN, NCH_MAIN)],
                    dst_v.at[pl.ds(0, NCH_MAIN)])

    @pl.when(wid < NCH_TAIL)
    def _():
        pltpu.sync_copy(src_hbm.at[NW * NCH_MAIN + wid], src_v.at[NCH_MAIN])
        pltpu.sync_copy(dst_hbm.at[NW * NCH_MAIN + wid], dst_v.at[NCH_MAIN])

    base_ch = wid * NCH_MAIN
    lanes129 = lax.iota(jnp.int32, L) * (CHUNK + 1)

    def fire(j, rows_ref, ep_ref, gsem, esem):
        pltpu.async_copy(ep_hbm.at[base_ch + j], ep_ref, esem)
        pltpu.async_copy(p_hbm.at[src_v.at[j]], rows_ref, gsem)

    def wait_slot(rows_ref, ep_ref, gsem, esem):
        pltpu.make_async_copy(ep_hbm.at[0], ep_ref, esem).wait()
        pltpu.make_async_copy(p_hbm.at[pl.ds(0, CHUNK)], rows_ref,
                              gsem).wait()

    def compute_scatter(j, rows_ref, ep_ref, ept_ref):
        for o in range(L):
            for b in range(CHUNK // L):
                ept_ref[pl.ds((CHUNK + 1) * o + L * b, L)] = (
                    ep_ref[pl.ds(CHUNK * o + L * b, L)])

        def edge(i, c2):
            epcol = plsc.load_gather(ept_ref, [lanes129 + i])
            rows_ref[i] = jnp.maximum(rows_ref[i] + epcol, 0.0)
            return c2
        lax.fori_loop(0, CHUNK, edge, 0, unroll=8)
        pltpu.sync_copy(rows_ref, acc.at[dst_v.at[j]], add=True)

    fire(0, rows0, ep0, gsem0, esem0)

    def body(j, carry):
        def proc(rq, eq, et, gs, es, ro, eo, go, eso):
            wait_slot(rq, eq, gs, es)

            @pl.when(j + 1 < NCH_MAIN)
            def _():
                fire(j + 1, ro, eo, go, eso)
            compute_scatter(j, rq, eq, et)

        @pl.when(j % 2 == 0)
        def _():
            proc(rows0, ep0, ept0, gsem0, esem0, rows1, ep1, gsem1, esem1)

        @pl.when(j % 2 == 1)
        def _():
            proc(rows1, ep1, ept1, gsem1, esem1, rows0, ep0, gsem0, esem0)
        return carry

    lax.fori_loop(0, NCH_MAIN, body, 0)

    @pl.when(wid < NCH_TAIL)
    def _():
        pltpu.sync_copy(ep_hbm.at[NW * NCH_MAIN + wid], ep0)
        pltpu.async_copy(p_hbm.at[src_v.at[NCH_MAIN]], rows0, gsem0).wait()
        compute_scatter(NCH_MAIN, rows0, ep0, ept0)

    plsc.subcore_barrier()

    pltpu.sync_copy(acc.at[pl.ds(s * ZROWS, ZROWS)],
                    out_hbm.at[c, pl.ds(s * ZROWS, ZROWS)])


def _make_sc_kernel():
    mesh = plsc.VectorSubcoreMesh(core_axis_name="c", subcore_axis_name="s",
                                  num_cores=NC, num_subcores=NS)
    return pl.kernel(
        _sc_edge_body,
        out_type=jax.ShapeDtypeStruct((NC, ACC_ROWS, L), jnp.float32),
        mesh=mesh,
        compiler_params=pltpu.CompilerParams(use_tc_tiling_on_sc=False,
                                             needs_layout_passes=False),
        scratch_types=[
            pltpu.VMEM((NCH_MAIN + 1, CHUNK), jnp.int32),
            pltpu.VMEM((NCH_MAIN + 1, CHUNK), jnp.int32),
            pltpu.VMEM((L * CHUNK,), jnp.float32),
            pltpu.VMEM((L * CHUNK,), jnp.float32),
            pltpu.VMEM((L * (CHUNK + 1),), jnp.float32),
            pltpu.VMEM((L * (CHUNK + 1),), jnp.float32),
            pltpu.VMEM((CHUNK, L), jnp.float32),
            pltpu.VMEM((CHUNK, L), jnp.float32),
            pltpu.VMEM((ZROWS, L), jnp.float32),
            pltpu.VMEM_SHARED((ACC_ROWS, L), jnp.float32),
            pltpu.SemaphoreType.DMA,
            pltpu.SemaphoreType.DMA,
            pltpu.SemaphoreType.DMA,
            pltpu.SemaphoreType.DMA,
        ],
    )


def kernel(node_representations, edges, edge_features, W1, b1, W2, W3, b3, W4):
    nodes = node_representations
    src = edges[0].astype(jnp.int32)
    dst = edges[1].astype(jnp.int32)
    pad = CHB * CHUNK - N_EDGES
    src_p = jnp.pad(src, (0, pad)).reshape(CHB, CHUNK)
    dst_p = jnp.pad(dst, (0, pad)).reshape(CHB, CHUNK)

    W1a = jnp.pad(W1[:D_EDGE], ((0, 0), (0, L - 10)))
    W1b = jnp.pad(W1[D_EDGE:], ((0, 0), (0, L - 10)))
    b1p = jnp.concatenate([b1, jnp.ones((1,), jnp.float32),
                           jnp.zeros((L - 11,), jnp.float32)]).reshape(L, 1)
    W2p = jnp.pad(W2, ((0, L - 10), (0, L - 10)))
    sel = jnp.zeros((L, 1), jnp.float32).at[10, 0].set(1.0)
    W3a = W3[:D_FEAT]
    W3b = jnp.pad(W3[D_FEAT:], ((0, L - 10), (0, 0)))
    b3r = b3.reshape(1, -1)

    proj_nodes = pl.pallas_call(
        _proj_nodes_body,
        out_shape=jax.ShapeDtypeStruct((N_NODES, L), jnp.float32),
    )
    P = proj_nodes(nodes, W1b)

    eft = edge_features.T
    proj_edges = pl.pallas_call(
        _proj_edges_body,
        grid=(NCH // CPB,),
        in_specs=[
            pl.BlockSpec((D_EDGE, CPB * CHUNK), lambda i: (0, i)),
            pl.BlockSpec((D_EDGE, L), lambda i: (0, 0)),
            pl.BlockSpec((L, 1), lambda i: (0, 0)),
        ],
        out_specs=pl.BlockSpec((CPB, L, CHUNK), lambda i: (i, 0, 0)),
        out_shape=jax.ShapeDtypeStruct((NCH, L, CHUNK), jnp.float32),
    )
    Ep = proj_edges(eft, W1a, b1p)

    parts = _make_sc_kernel()(src_p, dst_p, Ep.reshape(NCH, L * CHUNK), P)

    update = pl.pallas_call(
        _update_body,
        out_shape=jax.ShapeDtypeStruct((N_NODES, 10), jnp.float32),
    )
    return update(nodes, parts[0, :N_NODES], parts[1, :N_NODES], W2p, sel,
                  W3a, W3b, b3r, W4)


# speedup vs baseline: 12.8293x; 1.0031x over previous
import jax
import jax.numpy as jnp
from jax import lax
from jax.experimental import pallas as pl
from jax.experimental.pallas import tpu as pltpu
from jax.experimental.pallas import tpu_sc as plsc

N_NODES = 10000
N_EDGES = 320000
D_FEAT = 128
D_EDGE = 16
L = 16
NC = 2
NS = 16
NW = NC * NS
CHUNK = 128
NCH = N_EDGES // CHUNK
NCH_MAIN = NCH // NW
NCH_TAIL = NCH - NCH_MAIN * NW
CHB = NW * (NCH_MAIN + 1)
ACC_ROWS = 10240
ZROWS = ACC_ROWS // NS


def _proj_nodes_body(nodes_ref, w_ref, out_ref):
    out_ref[...] = jnp.dot(nodes_ref[...], w_ref[...],
                           preferred_element_type=jnp.float32)


CPB = 125


def _proj_edges_body(eft_ref, w_ref, b_ref, out_ref):
    ept = jax.lax.dot_general(w_ref[...], eft_ref[...],
                              (((0,), (0,)), ((), ())),
                              preferred_element_type=jnp.float32) + b_ref[...]
    for i in range(CPB):
        out_ref[i] = ept[:, CHUNK * i:CHUNK * (i + 1)]


def _update_body(nodes_ref, h0_ref, h1_ref, w2_ref, sel_ref, w3a_ref,
                 w3b_ref, b3_ref, w4_ref, out_ref):
    hs = h0_ref[...] + h1_ref[...]
    cnt = jnp.maximum(jnp.dot(hs, sel_ref[...],
                              preferred_element_type=jnp.float32), 1.0)
    agg = jnp.dot(hs, w2_ref[...], preferred_element_type=jnp.float32) / cnt
    u = jnp.dot(nodes_ref[...], w3a_ref[...],
                preferred_element_type=jnp.float32)
    u = u + jnp.dot(agg, w3b_ref[...], preferred_element_type=jnp.float32)
    u = jnp.maximum(u + b3_ref[...], 0.0)
    out_ref[...] = jnp.dot(u, w4_ref[...], preferred_element_type=jnp.float32)


def _sc_edge_body(src_hbm, dst_hbm, ep_hbm, p_hbm, out_hbm,
                  src_v, dst_v, ep0, ep1, ept0, ept1, rows0, rows1, zbuf, acc,
                  gsem0, gsem1, esem0, esem1):
    c = lax.axis_index("c")
    s = lax.axis_index("s")
    wid = s * NC + c

    def zero_row(i, carry):
        zbuf[i] = jnp.zeros((L,), jnp.float32)
        return carry
    lax.fori_loop(0, ZROWS, zero_row, 0)
    pltpu.sync_copy(zbuf, acc.at[pl.ds(s * ZROWS, ZROWS)])
    plsc.subcore_barrier()

    base_ch = wid * NCH_MAIN
    pltpu.sync_copy(src_hbm.at[pl.ds(base_ch, NCH_MAIN)],
                    src_v.at[pl.ds(0, NCH_MAIN)])
    pltpu.sync_copy(dst_hbm.at[pl.ds(base_ch, NCH_MAIN)],
                    dst_v.at[pl.ds(0, NCH_MAIN)])

    @pl.when(wid < NCH_TAIL)
    def _():
        pltpu.sync_copy(src_hbm.at[NW * NCH_MAIN + wid], src_v.at[NCH_MAIN])
        pltpu.sync_copy(dst_hbm.at[NW * NCH_MAIN + wid], dst_v.at[NCH_MAIN])
    lanes129 = lax.iota(jnp.int32, L) * (CHUNK + 1)

    def fire(j, rows_ref, ep_ref, gsem, esem):
        pltpu.async_copy(ep_hbm.at[base_ch + j], ep_ref, esem)
        pltpu.async_copy(p_hbm.at[src_v.at[j]], rows_ref, gsem)

    def wait_slot(rows_ref, ep_ref, gsem, esem):
        pltpu.make_async_copy(ep_hbm.at[0], ep_ref, esem).wait()
        pltpu.make_async_copy(p_hbm.at[pl.ds(0, CHUNK)], rows_ref,
                              gsem).wait()

    def compute(rows_ref, ep_ref, ept_ref):
        for o in range(L):
            for b in range(CHUNK // L):
                ept_ref[pl.ds((CHUNK + 1) * o + L * b, L)] = (
                    ep_ref[pl.ds(CHUNK * o + L * b, L)])

        def edge(i, c2):
            epcol = plsc.load_gather(ept_ref, [lanes129 + i])
            rows_ref[i] = jnp.maximum(rows_ref[i] + epcol, 0.0)
            return c2
        lax.fori_loop(0, CHUNK, edge, 0, unroll=16)

    fire(0, rows0, ep0, gsem0, esem0)

    def body(j, carry):
        def proc(rq, eq, et, gs, es, ro, eo, go, eso):
            wait_slot(rq, eq, gs, es)

            @pl.when(j + 1 < NCH_MAIN)
            def _():
                fire(j + 1, ro, eo, go, eso)
            compute(rq, eq, et)
            pltpu.sync_copy(rq, acc.at[dst_v.at[j]], add=True)

        @pl.when(j % 2 == 0)
        def _():
            proc(rows0, ep0, ept0, gsem0, esem0, rows1, ep1, gsem1, esem1)

        @pl.when(j % 2 == 1)
        def _():
            proc(rows1, ep1, ept1, gsem1, esem1, rows0, ep0, gsem0, esem0)
        return carry

    lax.fori_loop(0, NCH_MAIN, body, 0)

    @pl.when(wid < NCH_TAIL)
    def _():
        pltpu.sync_copy(ep_hbm.at[NW * NCH_MAIN + wid], ep0)
        pltpu.async_copy(p_hbm.at[src_v.at[NCH_MAIN]], rows0, gsem0).wait()
        compute(rows0, ep0, ept0)
        pltpu.sync_copy(rows0, acc.at[dst_v.at[NCH_MAIN]], add=True)

    plsc.subcore_barrier()

    pltpu.sync_copy(acc.at[pl.ds(s * ZROWS, ZROWS)],
                    out_hbm.at[c, pl.ds(s * ZROWS, ZROWS)])


def _make_sc_kernel():
    mesh = plsc.VectorSubcoreMesh(core_axis_name="c", subcore_axis_name="s",
                                  num_cores=NC, num_subcores=NS)
    return pl.kernel(
        _sc_edge_body,
        out_type=jax.ShapeDtypeStruct((NC, ACC_ROWS, L), jnp.float32),
        mesh=mesh,
        compiler_params=pltpu.CompilerParams(use_tc_tiling_on_sc=False,
                                             needs_layout_passes=False),
        scratch_types=[
            pltpu.VMEM((NCH_MAIN + 1, CHUNK), jnp.int32),
            pltpu.VMEM((NCH_MAIN + 1, CHUNK), jnp.int32),
            pltpu.VMEM((L * CHUNK,), jnp.float32),
            pltpu.VMEM((L * CHUNK,), jnp.float32),
            pltpu.VMEM((L * (CHUNK + 1),), jnp.float32),
            pltpu.VMEM((L * (CHUNK + 1),), jnp.float32),
            pltpu.VMEM((CHUNK, L), jnp.float32),
            pltpu.VMEM((CHUNK, L), jnp.float32),
            pltpu.VMEM((ZROWS, L), jnp.float32),
            pltpu.VMEM_SHARED((ACC_ROWS, L), jnp.float32),
            pltpu.SemaphoreType.DMA,
            pltpu.SemaphoreType.DMA,
            pltpu.SemaphoreType.DMA,
            pltpu.SemaphoreType.DMA,
        ],
    )


def kernel(node_representations, edges, edge_features, W1, b1, W2, W3, b3, W4):
    nodes = node_representations
    e32 = edges.astype(jnp.int32)
    src_p = e32[0].reshape(NCH, CHUNK)
    dst_p = e32[1].reshape(NCH, CHUNK)

    W1a = jnp.pad(W1[:D_EDGE], ((0, 0), (0, L - 10)))
    W1b = jnp.pad(W1[D_EDGE:], ((0, 0), (0, L - 10)))
    b1p = jnp.concatenate([b1, jnp.ones((1,), jnp.float32),
                           jnp.zeros((L - 11,), jnp.float32)]).reshape(L, 1)
    W2p = jnp.pad(W2, ((0, L - 10), (0, L - 10)))
    sel = jnp.zeros((L, 1), jnp.float32).at[10, 0].set(1.0)
    W3a = W3[:D_FEAT]
    W3b = jnp.pad(W3[D_FEAT:], ((0, L - 10), (0, 0)))
    b3r = b3.reshape(1, -1)

    proj_nodes = pl.pallas_call(
        _proj_nodes_body,
        out_shape=jax.ShapeDtypeStruct((N_NODES, L), jnp.float32),
    )
    P = proj_nodes(nodes, W1b)

    eft = edge_features.T
    proj_edges = pl.pallas_call(
        _proj_edges_body,
        grid=(NCH // CPB,),
        in_specs=[
            pl.BlockSpec((D_EDGE, CPB * CHUNK), lambda i: (0, i)),
            pl.BlockSpec((D_EDGE, L), lambda i: (0, 0)),
            pl.BlockSpec((L, 1), lambda i: (0, 0)),
        ],
        out_specs=pl.BlockSpec((CPB, L, CHUNK), lambda i: (i, 0, 0)),
        out_shape=jax.ShapeDtypeStruct((NCH, L, CHUNK), jnp.float32),
    )
    Ep = proj_edges(eft, W1a, b1p)

    parts = _make_sc_kernel()(src_p, dst_p, Ep.reshape(NCH, L * CHUNK), P)

    update = pl.pallas_call(
        _update_body,
        out_shape=jax.ShapeDtypeStruct((N_NODES, 10), jnp.float32),
    )
    return update(nodes, parts[0, :N_NODES], parts[1, :N_NODES], W2p, sel,
                  W3a, W3b, b3r, W4)


# speedup vs baseline: 12.8463x; 1.0013x over previous
import jax
import jax.numpy as jnp
from jax import lax
from jax.experimental import pallas as pl
from jax.experimental.pallas import tpu as pltpu
from jax.experimental.pallas import tpu_sc as plsc

N_NODES = 10000
N_EDGES = 320000
D_FEAT = 128
D_EDGE = 16
L = 16
NC = 2
NS = 16
NW = NC * NS
CHUNK = 128
NCH = N_EDGES // CHUNK
NCH_MAIN = NCH // NW
NCH_TAIL = NCH - NCH_MAIN * NW
CHB = NW * (NCH_MAIN + 1)
ACC_ROWS = 10240
ZROWS = ACC_ROWS // NS


def _proj_nodes_body(nodes_ref, w_ref, out_ref):
    out_ref[...] = jnp.dot(nodes_ref[...], w_ref[...],
                           preferred_element_type=jnp.float32)


CPB = 125


def _proj_edges_body(eft_ref, w_ref, b_ref, out_ref):
    ept = jax.lax.dot_general(w_ref[...], eft_ref[...],
                              (((0,), (0,)), ((), ())),
                              preferred_element_type=jnp.float32) + b_ref[...]
    for i in range(CPB):
        out_ref[i] = ept[:, CHUNK * i:CHUNK * (i + 1)]


def _update_a_body(nodes_ref, w3a_ref, b3_ref, out_ref):
    out_ref[...] = jnp.dot(nodes_ref[...], w3a_ref[...],
                           preferred_element_type=jnp.float32) + b3_ref[...]


def _update_b_body(u1_ref, h0_ref, h1_ref, w2_ref, sel_ref,
                   w3b_ref, w4_ref, out_ref):
    hs = h0_ref[...] + h1_ref[...]
    cnt = jnp.maximum(jnp.dot(hs, sel_ref[...],
                              preferred_element_type=jnp.float32), 1.0)
    agg = jnp.dot(hs, w2_ref[...], preferred_element_type=jnp.float32) / cnt
    u = u1_ref[...] + jnp.dot(agg, w3b_ref[...],
                              preferred_element_type=jnp.float32)
    u = jnp.maximum(u, 0.0)
    out_ref[...] = jnp.dot(u, w4_ref[...], preferred_element_type=jnp.float32)


def _sc_edge_body(src_hbm, dst_hbm, ep_hbm, p_hbm, out_hbm,
                  src_v, dst_v, ep0, ep1, ep2, ept0, ept1, ept2,
                  rows0, rows1, rows2, zbuf, acc,
                  gsem0, gsem1, gsem2, esem0, esem1, esem2):
    c = lax.axis_index("c")
    s = lax.axis_index("s")
    wid = s * NC + c

    def zero_row(i, carry):
        zbuf[i] = jnp.zeros((L,), jnp.float32)
        return carry
    lax.fori_loop(0, ZROWS, zero_row, 0)
    pltpu.sync_copy(zbuf, acc.at[pl.ds(s * ZROWS, ZROWS)])
    plsc.subcore_barrier()

    base_ch = wid * NCH_MAIN
    pltpu.sync_copy(src_hbm.at[pl.ds(base_ch, NCH_MAIN)],
                    src_v.at[pl.ds(0, NCH_MAIN)])
    pltpu.sync_copy(dst_hbm.at[pl.ds(base_ch, NCH_MAIN)],
                    dst_v.at[pl.ds(0, NCH_MAIN)])

    @pl.when(wid < NCH_TAIL)
    def _():
        pltpu.sync_copy(src_hbm.at[NW * NCH_MAIN + wid], src_v.at[NCH_MAIN])
        pltpu.sync_copy(dst_hbm.at[NW * NCH_MAIN + wid], dst_v.at[NCH_MAIN])
    lanes129 = lax.iota(jnp.int32, L) * (CHUNK + 1)

    def fire(j, rows_ref, ep_ref, gsem, esem):
        pltpu.async_copy(ep_hbm.at[base_ch + j], ep_ref, esem)
        pltpu.async_copy(p_hbm.at[src_v.at[j]], rows_ref, gsem)

    def wait_slot(rows_ref, ep_ref, gsem, esem):
        pltpu.make_async_copy(ep_hbm.at[0], ep_ref, esem).wait()
        pltpu.make_async_copy(p_hbm.at[pl.ds(0, CHUNK)], rows_ref,
                              gsem).wait()

    def compute(rows_ref, ep_ref, ept_ref):
        for o in range(L):
            for b in range(CHUNK // L):
                ept_ref[pl.ds((CHUNK + 1) * o + L * b, L)] = (
                    ep_ref[pl.ds(CHUNK * o + L * b, L)])

        def edge(i, c2):
            epcol = plsc.load_gather(ept_ref, [lanes129 + i])
            rows_ref[i] = jnp.maximum(rows_ref[i] + epcol, 0.0)
            return c2
        lax.fori_loop(0, CHUNK, edge, 0, unroll=16)

    fire(0, rows0, ep0, gsem0, esem0)
    fire(1, rows1, ep1, gsem1, esem1)

    def body(j, carry):
        def proc(rq, eq, et, gs, es, r2, e2, g2, es2):
            wait_slot(rq, eq, gs, es)

            @pl.when(j + 2 < NCH_MAIN)
            def _():
                fire(j + 2, r2, e2, g2, es2)
            compute(rq, eq, et)
            pltpu.sync_copy(rq, acc.at[dst_v.at[j]], add=True)

        @pl.when(j % 3 == 0)
        def _():
            proc(rows0, ep0, ept0, gsem0, esem0, rows2, ep2, gsem2, esem2)

        @pl.when(j % 3 == 1)
        def _():
            proc(rows1, ep1, ept1, gsem1, esem1, rows0, ep0, gsem0, esem0)

        @pl.when(j % 3 == 2)
        def _():
            proc(rows2, ep2, ept2, gsem2, esem2, rows1, ep1, gsem1, esem1)
        return carry

    lax.fori_loop(0, NCH_MAIN, body, 0)

    @pl.when(wid < NCH_TAIL)
    def _():
        pltpu.sync_copy(ep_hbm.at[NW * NCH_MAIN + wid], ep0)
        pltpu.async_copy(p_hbm.at[src_v.at[NCH_MAIN]], rows0, gsem0).wait()
        compute(rows0, ep0, ept0)
        pltpu.sync_copy(rows0, acc.at[dst_v.at[NCH_MAIN]], add=True)

    plsc.subcore_barrier()

    pltpu.sync_copy(acc.at[pl.ds(s * ZROWS, ZROWS)],
                    out_hbm.at[c, pl.ds(s * ZROWS, ZROWS)])


def _make_sc_kernel():
    mesh = plsc.VectorSubcoreMesh(core_axis_name="c", subcore_axis_name="s",
                                  num_cores=NC, num_subcores=NS)
    return pl.kernel(
        _sc_edge_body,
        out_type=jax.ShapeDtypeStruct((NC, ACC_ROWS, L), jnp.float32),
        mesh=mesh,
        compiler_params=pltpu.CompilerParams(use_tc_tiling_on_sc=False,
                                             needs_layout_passes=False),
        scratch_types=[
            pltpu.VMEM((NCH_MAIN + 1, CHUNK), jnp.int32),
            pltpu.VMEM((NCH_MAIN + 1, CHUNK), jnp.int32),
            pltpu.VMEM((L * CHUNK,), jnp.float32),
            pltpu.VMEM((L * CHUNK,), jnp.float32),
            pltpu.VMEM((L * CHUNK,), jnp.float32),
            pltpu.VMEM((L * (CHUNK + 1),), jnp.float32),
            pltpu.VMEM((L * (CHUNK + 1),), jnp.float32),
            pltpu.VMEM((L * (CHUNK + 1),), jnp.float32),
            pltpu.VMEM((CHUNK, L), jnp.float32),
            pltpu.VMEM((CHUNK, L), jnp.float32),
            pltpu.VMEM((CHUNK, L), jnp.float32),
            pltpu.VMEM((ZROWS, L), jnp.float32),
            pltpu.VMEM_SHARED((ACC_ROWS, L), jnp.float32),
            pltpu.SemaphoreType.DMA,
            pltpu.SemaphoreType.DMA,
            pltpu.SemaphoreType.DMA,
            pltpu.SemaphoreType.DMA,
            pltpu.SemaphoreType.DMA,
            pltpu.SemaphoreType.DMA,
        ],
    )


def kernel(node_representations, edges, edge_features, W1, b1, W2, W3, b3, W4):
    nodes = node_representations
    e32 = edges.astype(jnp.int32)
    src_p = e32[0].reshape(NCH, CHUNK)
    dst_p = e32[1].reshape(NCH, CHUNK)

    W1a = jnp.pad(W1[:D_EDGE], ((0, 0), (0, L - 10)))
    W1b = jnp.pad(W1[D_EDGE:], ((0, 0), (0, L - 10)))
    b1p = jnp.concatenate([b1, jnp.ones((1,), jnp.float32),
                           jnp.zeros((L - 11,), jnp.float32)]).reshape(L, 1)
    W2p = jnp.pad(W2, ((0, L - 10), (0, L - 10)))
    sel = jnp.zeros((L, 1), jnp.float32).at[10, 0].set(1.0)
    W3a = W3[:D_FEAT]
    W3b = jnp.pad(W3[D_FEAT:], ((0, L - 10), (0, 0)))
    b3r = b3.reshape(1, -1)

    proj_nodes = pl.pallas_call(
        _proj_nodes_body,
        out_shape=jax.ShapeDtypeStruct((N_NODES, L), jnp.float32),
    )
    P = proj_nodes(nodes, W1b)

    eft = edge_features.T
    proj_edges = pl.pallas_call(
        _proj_edges_body,
        grid=(NCH // CPB,),
        in_specs=[
            pl.BlockSpec((D_EDGE, CPB * CHUNK), lambda i: (0, i)),
            pl.BlockSpec((D_EDGE, L), lambda i: (0, 0)),
            pl.BlockSpec((L, 1), lambda i: (0, 0)),
        ],
        out_specs=pl.BlockSpec((CPB, L, CHUNK), lambda i: (i, 0, 0)),
        out_shape=jax.ShapeDtypeStruct((NCH, L, CHUNK), jnp.float32),
    )
    Ep = proj_edges(eft, W1a, b1p)

    parts = _make_sc_kernel()(src_p, dst_p, Ep.reshape(NCH, L * CHUNK), P)

    update_a = pl.pallas_call(
        _update_a_body,
        out_shape=jax.ShapeDtypeStruct((N_NODES, 20), jnp.float32),
    )
    U1 = update_a(nodes, W3a, b3r)

    update_b = pl.pallas_call(
        _update_b_body,
        out_shape=jax.ShapeDtypeStruct((N_NODES, 10), jnp.float32),
    )
    return update_b(U1, parts[0, :N_NODES], parts[1, :N_NODES], W2p, sel,
                    W3b, W4)


# speedup vs baseline: 13.8738x; 1.0800x over previous
import jax
import jax.numpy as jnp
from jax import lax
from jax.experimental import pallas as pl
from jax.experimental.pallas import tpu as pltpu
from jax.experimental.pallas import tpu_sc as plsc

N_NODES = 10000
N_EDGES = 320000
D_FEAT = 128
D_EDGE = 16
L = 16
NC = 2
NS = 16
NW = NC * NS
CHUNK = 128
NCH = N_EDGES // CHUNK
NCH_MAIN = NCH // NW
NCH_TAIL = NCH - NCH_MAIN * NW
CHB = NW * (NCH_MAIN + 1)
ACC_ROWS = 10240
ZROWS = ACC_ROWS // NS


def _proj_nodes_body(nodes_ref, w_ref, out_ref):
    out_ref[...] = jnp.dot(nodes_ref[...], w_ref[...],
                           preferred_element_type=jnp.float32)


CPB = 125


def _proj_edges_body(eft_ref, w_ref, b_ref, out_ref):
    ept = jax.lax.dot_general(w_ref[...], eft_ref[...],
                              (((0,), (0,)), ((), ())),
                              preferred_element_type=jnp.float32) + b_ref[...]
    for i in range(CPB):
        out_ref[i] = ept[:, CHUNK * i:CHUNK * (i + 1)]


def _update_a_body(nodes_ref, w3a_ref, b3_ref, out_ref):
    out_ref[...] = jnp.dot(nodes_ref[...], w3a_ref[...],
                           preferred_element_type=jnp.float32) + b3_ref[...]


def _update_b_body(u1_ref, h0_ref, h1_ref, w2_ref, sel_ref,
                   w3b_ref, w4_ref, out_ref):
    hs = h0_ref[...] + h1_ref[...]
    cnt = jnp.maximum(jnp.dot(hs, sel_ref[...],
                              preferred_element_type=jnp.float32), 1.0)
    agg = jnp.dot(hs, w2_ref[...], preferred_element_type=jnp.float32) / cnt
    u = u1_ref[...] + jnp.dot(agg, w3b_ref[...],
                              preferred_element_type=jnp.float32)
    u = jnp.maximum(u, 0.0)
    out_ref[...] = jnp.dot(u, w4_ref[...], preferred_element_type=jnp.float32)


def _sc_edge_body(e_hbm, ep_hbm, p_hbm, out_hbm,
                  src_v, dst_v, ep0, ep1, ep2, ept0, ept1, ept2,
                  rows0, rows1, rows2, zbuf, acc,
                  gsem0, gsem1, gsem2, esem0, esem1, esem2):
    c = lax.axis_index("c")
    s = lax.axis_index("s")
    wid = s * NC + c

    def zero_row(i, carry):
        zbuf[i] = jnp.zeros((L,), jnp.float32)
        return carry
    lax.fori_loop(0, ZROWS, zero_row, 0)
    pltpu.sync_copy(zbuf, acc.at[pl.ds(s * ZROWS, ZROWS)])
    plsc.subcore_barrier()

    base_ch = wid * NCH_MAIN
    pltpu.sync_copy(e_hbm.at[pl.ds(base_ch, NCH_MAIN), 0],
                    src_v.at[pl.ds(0, NCH_MAIN)])
    pltpu.sync_copy(e_hbm.at[pl.ds(base_ch, NCH_MAIN), 1],
                    dst_v.at[pl.ds(0, NCH_MAIN)])

    @pl.when(wid < NCH_TAIL)
    def _():
        pltpu.sync_copy(e_hbm.at[NW * NCH_MAIN + wid, 0], src_v.at[NCH_MAIN])
        pltpu.sync_copy(e_hbm.at[NW * NCH_MAIN + wid, 1], dst_v.at[NCH_MAIN])
    lanes129 = lax.iota(jnp.int32, L) * (CHUNK + 1)

    def fire(j, rows_ref, ep_ref, gsem, esem):
        pltpu.async_copy(ep_hbm.at[base_ch + j], ep_ref, esem)
        pltpu.async_copy(p_hbm.at[src_v.at[j]], rows_ref, gsem)

    def wait_slot(rows_ref, ep_ref, gsem, esem):
        pltpu.make_async_copy(ep_hbm.at[0], ep_ref, esem).wait()
        pltpu.make_async_copy(p_hbm.at[pl.ds(0, CHUNK)], rows_ref,
                              gsem).wait()

    def compute(rows_ref, ep_ref, ept_ref):
        for o in range(L):
            for b in range(CHUNK // L):
                ept_ref[pl.ds((CHUNK + 1) * o + L * b, L)] = (
                    ep_ref[pl.ds(CHUNK * o + L * b, L)])

        def edge(i, c2):
            epcol = plsc.load_gather(ept_ref, [lanes129 + i])
            rows_ref[i] = jnp.maximum(rows_ref[i] + epcol, 0.0)
            return c2
        lax.fori_loop(0, CHUNK, edge, 0, unroll=16)

    fire(0, rows0, ep0, gsem0, esem0)
    fire(1, rows1, ep1, gsem1, esem1)

    def body(j, carry):
        def proc(rq, eq, et, gs, es, r2, e2, g2, es2):
            wait_slot(rq, eq, gs, es)

            @pl.when(j + 2 < NCH_MAIN)
            def _():
                fire(j + 2, r2, e2, g2, es2)
            compute(rq, eq, et)
            pltpu.sync_copy(rq, acc.at[dst_v.at[j]], add=True)

        @pl.when(j % 3 == 0)
        def _():
            proc(rows0, ep0, ept0, gsem0, esem0, rows2, ep2, gsem2, esem2)

        @pl.when(j % 3 == 1)
        def _():
            proc(rows1, ep1, ept1, gsem1, esem1, rows0, ep0, gsem0, esem0)

        @pl.when(j % 3 == 2)
        def _():
            proc(rows2, ep2, ept2, gsem2, esem2, rows1, ep1, gsem1, esem1)
        return carry

    lax.fori_loop(0, NCH_MAIN, body, 0)

    @pl.when(wid < NCH_TAIL)
    def _():
        pltpu.sync_copy(ep_hbm.at[NW * NCH_MAIN + wid], ep0)
        pltpu.async_copy(p_hbm.at[src_v.at[NCH_MAIN]], rows0, gsem0).wait()
        compute(rows0, ep0, ept0)
        pltpu.sync_copy(rows0, acc.at[dst_v.at[NCH_MAIN]], add=True)

    plsc.subcore_barrier()

    pltpu.sync_copy(acc.at[pl.ds(s * ZROWS, ZROWS)],
                    out_hbm.at[c, pl.ds(s * ZROWS, ZROWS)])


def _make_sc_kernel():
    mesh = plsc.VectorSubcoreMesh(core_axis_name="c", subcore_axis_name="s",
                                  num_cores=NC, num_subcores=NS)
    return pl.kernel(
        _sc_edge_body,
        out_type=jax.ShapeDtypeStruct((NC, ACC_ROWS, L), jnp.float32),
        mesh=mesh,
        compiler_params=pltpu.CompilerParams(use_tc_tiling_on_sc=False,
                                             needs_layout_passes=False),
        scratch_types=[
            pltpu.VMEM((NCH_MAIN + 1, CHUNK), jnp.int32),
            pltpu.VMEM((NCH_MAIN + 1, CHUNK), jnp.int32),
            pltpu.VMEM((L * CHUNK,), jnp.float32),
            pltpu.VMEM((L * CHUNK,), jnp.float32),
            pltpu.VMEM((L * CHUNK,), jnp.float32),
            pltpu.VMEM((L * (CHUNK + 1),), jnp.float32),
            pltpu.VMEM((L * (CHUNK + 1),), jnp.float32),
            pltpu.VMEM((L * (CHUNK + 1),), jnp.float32),
            pltpu.VMEM((CHUNK, L), jnp.float32),
            pltpu.VMEM((CHUNK, L), jnp.float32),
            pltpu.VMEM((CHUNK, L), jnp.float32),
            pltpu.VMEM((ZROWS, L), jnp.float32),
            pltpu.VMEM_SHARED((ACC_ROWS, L), jnp.float32),
            pltpu.SemaphoreType.DMA,
            pltpu.SemaphoreType.DMA,
            pltpu.SemaphoreType.DMA,
            pltpu.SemaphoreType.DMA,
            pltpu.SemaphoreType.DMA,
            pltpu.SemaphoreType.DMA,
        ],
    )


def kernel(node_representations, edges, edge_features, W1, b1, W2, W3, b3, W4):
    nodes = node_representations
    ec = edges.astype(jnp.int32).reshape(2, NCH, CHUNK).transpose(1, 0, 2)

    W1a = jnp.pad(W1[:D_EDGE], ((0, 0), (0, L - 10)))
    W1b = jnp.pad(W1[D_EDGE:], ((0, 0), (0, L - 10)))
    b1p = jnp.concatenate([b1, jnp.ones((1,), jnp.float32),
                           jnp.zeros((L - 11,), jnp.float32)]).reshape(L, 1)
    W2p = jnp.pad(W2, ((0, L - 10), (0, L - 10)))
    sel = jnp.zeros((L, 1), jnp.float32).at[10, 0].set(1.0)
    W3a = W3[:D_FEAT]
    W3b = jnp.pad(W3[D_FEAT:], ((0, L - 10), (0, 0)))
    b3r = b3.reshape(1, -1)

    proj_nodes = pl.pallas_call(
        _proj_nodes_body,
        out_shape=jax.ShapeDtypeStruct((N_NODES, L), jnp.float32),
    )
    P = proj_nodes(nodes, W1b)

    eft = edge_features.T
    proj_edges = pl.pallas_call(
        _proj_edges_body,
        grid=(NCH // CPB,),
        in_specs=[
            pl.BlockSpec((D_EDGE, CPB * CHUNK), lambda i: (0, i)),
            pl.BlockSpec((D_EDGE, L), lambda i: (0, 0)),
            pl.BlockSpec((L, 1), lambda i: (0, 0)),
        ],
        out_specs=pl.BlockSpec((CPB, L, CHUNK), lambda i: (i, 0, 0)),
        out_shape=jax.ShapeDtypeStruct((NCH, L, CHUNK), jnp.float32),
    )
    Ep = proj_edges(eft, W1a, b1p)

    parts = _make_sc_kernel()(ec, Ep.reshape(NCH, L * CHUNK), P)

    update_a = pl.pallas_call(
        _update_a_body,
        out_shape=jax.ShapeDtypeStruct((N_NODES, 20), jnp.float32),
    )
    U1 = update_a(nodes, W3a, b3r)

    update_b = pl.pallas_call(
        _update_b_body,
        out_shape=jax.ShapeDtypeStruct((N_NODES, 10), jnp.float32),
    )
    return update_b(U1, parts[0, :N_NODES], parts[1, :N_NODES], W2p, sel,
                    W3b, W4)
